# baseline (device time: 55002 ns/iter reference)
import jax
import jax.numpy as jnp
from jax import lax
from jax.experimental import pallas as pl
from jax.experimental.pallas import tpu as pltpu

N_DEV = 4


def kernel(x, w_mat, scale_x, scale_w):
    M, k_per = x.shape
    K, N = w_mat.shape
    m_per = M // N_DEV

    x8 = x.astype(jnp.float8_e5m2).reshape(N_DEV, m_per, k_per)
    w8 = w_mat.astype(jnp.float8_e5m2).reshape(N_DEV, k_per, N)
    s = (scale_x * scale_w).reshape(1, 1)

    def body(x_ref, w_ref, s_ref, out_ref, comm_ref, send_sems, recv_sems):
        my = lax.axis_index("i")

        barrier = pltpu.get_barrier_semaphore()
        for d in range(1, N_DEV):
            pl.semaphore_signal(
                barrier, inc=1,
                device_id=((my + d) % N_DEV,),
                device_id_type=pltpu.DeviceIdType.MESH,
            )
        pl.semaphore_wait(barrier, N_DEV - 1)

        sends = []
        for d in range(1, N_DEV):
            tgt = (my + d) % N_DEV
            rdma = pltpu.make_async_remote_copy(
                src_ref=x_ref.at[tgt],
                dst_ref=comm_ref.at[d - 1],
                send_sem=send_sems.at[d - 1],
                recv_sem=recv_sems.at[d - 1],
                device_id=(tgt,),
                device_id_type=pltpu.DeviceIdType.MESH,
            )
            rdma.start()
            sends.append(rdma)

        out_ref[...] = jnp.dot(
            x_ref[my], w_ref[my], preferred_element_type=jnp.float32
        )

        for d in (1, 3, 2):
            src = (my - d) % N_DEV
            recv = pltpu.make_async_remote_copy(
                src_ref=comm_ref.at[d - 1],
                dst_ref=comm_ref.at[d - 1],
                send_sem=send_sems.at[d - 1],
                recv_sem=recv_sems.at[d - 1],
                device_id=(src,),
                device_id_type=pltpu.DeviceIdType.MESH,
            )
            recv.wait_recv()
            out_ref[...] += jnp.dot(
                comm_ref[d - 1], w_ref[src],
                preferred_element_type=jnp.float32,
            )

        out_ref[...] = out_ref[...] * s_ref[0, 0]

        for rdma in sends:
            rdma.wait_send()

    return pl.pallas_call(
        body,
        out_shape=jax.ShapeDtypeStruct((m_per, N), jnp.float32),
        in_specs=[
            pl.BlockSpec(memory_space=pltpu.VMEM),
            pl.BlockSpec(memory_space=pltpu.VMEM),
            pl.BlockSpec(memory_space=pltpu.SMEM),
        ],
        out_specs=pl.BlockSpec(memory_space=pltpu.VMEM),
        scratch_shapes=[
            pltpu.VMEM((N_DEV - 1, m_per, k_per), jnp.float8_e5m2),
            pltpu.SemaphoreType.DMA((N_DEV - 1,)),
            pltpu.SemaphoreType.DMA((N_DEV - 1,)),
        ],
        compiler_params=pltpu.CompilerParams(collective_id=0),
    )(x8, w8, s)


# device time: 48754 ns/iter; 1.1282x vs baseline; 1.1282x over previous
import jax
import jax.numpy as jnp
from jax import lax
from jax.experimental import pallas as pl
from jax.experimental.pallas import tpu as pltpu

N_DEV = 4


def kernel(x, w_mat, scale_x, scale_w):
    M, k_per = x.shape
    K, N = w_mat.shape
    m_per = M // N_DEV

    x32 = x.reshape(N_DEV, m_per, k_per)
    w32 = w_mat.reshape(N_DEV, k_per, N)
    s = (scale_x * scale_w).reshape(1, 1)

    def body(x_ref, w_hbm, s_ref, out_ref,
             x8_ref, comm_ref, wbuf, w8buf,
             send_sems, recv_sems, w_sems):
        my = lax.axis_index("i")

        x8_ref[...] = x_ref[...].astype(jnp.float8_e5m2)

        barrier = pltpu.get_barrier_semaphore()
        for d in range(1, N_DEV):
            pl.semaphore_signal(
                barrier, inc=1,
                device_id=((my + d) % N_DEV,),
                device_id_type=pltpu.DeviceIdType.MESH,
            )
        pl.semaphore_wait(barrier, N_DEV - 1)

        sends = []
        for d in range(1, N_DEV):
            tgt = (my + d) % N_DEV
            rdma = pltpu.make_async_remote_copy(
                src_ref=x8_ref.at[tgt],
                dst_ref=comm_ref.at[d - 1],
                send_sem=send_sems.at[d - 1],
                recv_sem=recv_sems.at[d - 1],
                device_id=(tgt,),
                device_id_type=pltpu.DeviceIdType.MESH,
            )
            rdma.start()
            sends.append(rdma)

        w_order = [my] + [(my - d) % N_DEV for d in (1, 3, 2)]
        w_dmas = []
        for k, blk in enumerate(w_order[:2]):
            dma = pltpu.make_async_copy(w_hbm.at[blk], wbuf.at[k % 2],
                                        w_sems.at[k % 2])
            dma.start()
            w_dmas.append(dma)

        def consume_w(k):
            slot = k % 2
            w_dmas[k].wait()
            w8buf[slot] = wbuf[slot].astype(jnp.float8_e5m2)
            if k + 2 < N_DEV:
                dma = pltpu.make_async_copy(w_hbm.at[w_order[k + 2]],
                                            wbuf.at[slot], w_sems.at[slot])
                dma.start()
                w_dmas.append(dma)
            return slot

        slot = consume_w(0)
        out_ref[...] = jnp.dot(
            x8_ref[my], w8buf[slot], preferred_element_type=jnp.float32
        )

        for k, d in enumerate((1, 3, 2), start=1):
            recv = pltpu.make_async_remote_copy(
                src_ref=comm_ref.at[d - 1],
                dst_ref=comm_ref.at[d - 1],
                send_sem=send_sems.at[d - 1],
                recv_sem=recv_sems.at[d - 1],
                device_id=(my,),
                device_id_type=pltpu.DeviceIdType.MESH,
            )
            recv.wait_recv()
            slot = consume_w(k)
            acc = out_ref[...] + jnp.dot(
                comm_ref[d - 1], w8buf[slot],
                preferred_element_type=jnp.float32,
            )
            out_ref[...] = acc * s_ref[0, 0] if k == N_DEV - 1 else acc

        for rdma in sends:
            rdma.wait_send()

    return pl.pallas_call(
        body,
        out_shape=jax.ShapeDtypeStruct((m_per, N), jnp.float32),
        in_specs=[
            pl.BlockSpec(memory_space=pltpu.VMEM),
            pl.BlockSpec(memory_space=pltpu.MemorySpace.HBM),
            pl.BlockSpec(memory_space=pltpu.SMEM),
        ],
        out_specs=pl.BlockSpec(memory_space=pltpu.VMEM),
        scratch_shapes=[
            pltpu.VMEM((N_DEV, m_per, k_per), jnp.float8_e5m2),
            pltpu.VMEM((N_DEV - 1, m_per, k_per), jnp.float8_e5m2),
            pltpu.VMEM((2, k_per, N), jnp.float32),
            pltpu.VMEM((2, k_per, N), jnp.float8_e5m2),
            pltpu.SemaphoreType.DMA((N_DEV - 1,)),
            pltpu.SemaphoreType.DMA((N_DEV - 1,)),
            pltpu.SemaphoreType.DMA((2,)),
        ],
        compiler_params=pltpu.CompilerParams(
            collective_id=0, vmem_limit_bytes=100 * 1024 * 1024,
        ),
    )(x32, w32, s)


# device time: 48582 ns/iter; 1.1321x vs baseline; 1.0035x over previous
import os

import jax
import jax.numpy as jnp
from jax import lax
from jax.experimental import pallas as pl
from jax.experimental.pallas import tpu as pltpu

N_DEV = 4

_VARIANT = os.environ.get("KVAR", "full")
_DO_COMM = _VARIANT != "nocomm"
_DO_DOT = _VARIANT != "nodot"
_DO_W = _VARIANT != "now"


def kernel(x, w_mat, scale_x, scale_w):
    M, k_per = x.shape
    K, N = w_mat.shape
    m_per = M // N_DEV

    x32 = x.reshape(N_DEV, m_per, k_per)
    w32 = w_mat.reshape(N_DEV, k_per, N)
    s = (scale_x * scale_w).reshape(1, 1)

    def body(x_ref, w_hbm, s_ref, out_ref,
             x8_ref, comm_ref, wbuf, w8buf,
             send_sems, recv_sems, w_sems):
        my = lax.axis_index("i")

        x8_ref[...] = x_ref[...].astype(jnp.float8_e5m2)

        sends = []
        if _DO_COMM:
            barrier = pltpu.get_barrier_semaphore()
            for d in range(1, N_DEV):
                pl.semaphore_signal(
                    barrier, inc=1,
                    device_id=((my + d) % N_DEV,),
                    device_id_type=pltpu.DeviceIdType.MESH,
                )
            pl.semaphore_wait(barrier, N_DEV - 1)

            for d in range(1, N_DEV):
                tgt = (my + d) % N_DEV
                rdma = pltpu.make_async_remote_copy(
                    src_ref=x8_ref.at[tgt],
                    dst_ref=comm_ref.at[d - 1],
                    send_sem=send_sems.at[d - 1],
                    recv_sem=recv_sems.at[d - 1],
                    device_id=(tgt,),
                    device_id_type=pltpu.DeviceIdType.MESH,
                )
                rdma.start()
                sends.append(rdma)

        w_order = [my] + [(my - d) % N_DEV for d in (1, 3, 2)]
        w_dmas = []
        if _DO_W:
            for k, blk in enumerate(w_order[:2]):
                dma = pltpu.make_async_copy(w_hbm.at[blk], wbuf.at[k % 2],
                                            w_sems.at[k % 2])
                dma.start()
                w_dmas.append(dma)

        def consume_w(k):
            slot = k % 2
            if not _DO_W:
                return slot
            w_dmas[k].wait()
            w8buf[slot] = wbuf[slot].astype(jnp.float8_e5m2)
            if k + 2 < N_DEV:
                dma = pltpu.make_async_copy(w_hbm.at[w_order[k + 2]],
                                            wbuf.at[slot], w_sems.at[slot])
                dma.start()
                w_dmas.append(dma)
            return slot

        slot = consume_w(0)
        if _DO_DOT:
            out_ref[...] = jnp.dot(
                x8_ref[my], w8buf[slot], preferred_element_type=jnp.float32
            )
        else:
            out_ref[...] = w8buf[slot].astype(jnp.float32)
            out_ref[:, :k_per] += x8_ref[my].astype(jnp.float32)

        for k, d in enumerate((1, 3, 2), start=1):
            if _DO_COMM:
                recv = pltpu.make_async_remote_copy(
                    src_ref=comm_ref.at[d - 1],
                    dst_ref=comm_ref.at[d - 1],
                    send_sem=send_sems.at[d - 1],
                    recv_sem=recv_sems.at[d - 1],
                    device_id=(my,),
                    device_id_type=pltpu.DeviceIdType.MESH,
                )
                recv.wait_recv()
            slot = consume_w(k)
            if _DO_DOT:
                acc = out_ref[...] + jnp.dot(
                    comm_ref[d - 1], w8buf[slot],
                    preferred_element_type=jnp.float32,
                )
                out_ref[...] = acc * s_ref[0, 0] if k == N_DEV - 1 else acc
            else:
                out_ref[...] += w8buf[slot].astype(jnp.float32)
                out_ref[:, :k_per] += comm_ref[d - 1].astype(jnp.float32)

        for rdma in sends:
            rdma.wait_send()

    return pl.pallas_call(
        body,
        out_shape=jax.ShapeDtypeStruct((m_per, N), jnp.float32),
        in_specs=[
            pl.BlockSpec(memory_space=pltpu.VMEM),
            pl.BlockSpec(memory_space=pltpu.MemorySpace.HBM),
            pl.BlockSpec(memory_space=pltpu.SMEM),
        ],
        out_specs=pl.BlockSpec(memory_space=pltpu.VMEM),
        scratch_shapes=[
            pltpu.VMEM((N_DEV, m_per, k_per), jnp.float8_e5m2),
            pltpu.VMEM((N_DEV - 1, m_per, k_per), jnp.float8_e5m2),
            pltpu.VMEM((2, k_per, N), jnp.float32),
            pltpu.VMEM((2, k_per, N), jnp.float8_e5m2),
            pltpu.SemaphoreType.DMA((N_DEV - 1,)),
            pltpu.SemaphoreType.DMA((N_DEV - 1,)),
            pltpu.SemaphoreType.DMA((2,)),
        ],
        compiler_params=pltpu.CompilerParams(
            collective_id=0 if _DO_COMM else None,
            vmem_limit_bytes=100 * 1024 * 1024,
        ),
    )(x32, w32, s)
